# Initial kernel scaffold; baseline (speedup 1.0000x reference)
import jax
import jax.numpy as jnp
from jax.experimental import pallas as pl


def kernel(x, edge_index, edge_attr, batch, params):
    raise NotImplementedError("write your pallas kernel here")


# speedup vs baseline: 2.5061x; 2.5061x over previous
import functools

import jax
import jax.numpy as jnp
from jax import lax
from jax.experimental import pallas as pl
from jax.experimental.pallas import tpu as pltpu
from jax.experimental.pallas import tpu_sc as plsc

_N = 10000
_E = 640000
_D = 128
_FIN = 39
_FE = 10
_L = 3
_PREC = lax.Precision.DEFAULT


def _dot3(a, b):
    return jnp.dot(a.astype(jnp.bfloat16), b.astype(jnp.bfloat16),
                   preferred_element_type=jnp.float32, precision=_PREC)

_NC = 2
_NS = 16
_NW = _NC * _NS
_C = 128
_CH = -(-_E // (_NW * _C))
_EPAD = _NW * _CH * _C
_RPT = 632
_NPAD = _NS * _RPT


def _stride8(r):
    s4 = r[0:4] + r[4:8]
    s2 = s4[0:2] + s4[2:4]
    return s2[0:1] + s2[1:2]


def _bn_ref(ref):
    n, d = ref.shape

    def msum(i, acc):
        return acc + ref[pl.ds(pl.multiple_of(8 * i, 8), 8)]

    m = _stride8(lax.fori_loop(1, n // 8, msum, ref[0:8])) * (1.0 / n)

    def vsum(i, acc):
        td = ref[pl.ds(pl.multiple_of(8 * i, 8), 8)] - m
        return acc + td * td

    def vchunk(lo, hi):
        t0 = ref[pl.ds(8 * lo, 8)] - m
        return _stride8(lax.fori_loop(lo + 1, hi, vsum, t0 * t0))

    half = (n // 8) // 2
    v = (vchunk(0, half) + vchunk(half, n // 8)) * (1.0 / n)
    return (ref[...] - m) / jnp.sqrt(v + 1e-5)


def _enc_body(x_ref, w0_ref, b0_ref, w1_ref, b1_ref, o_ref, t_ref):
    h = _dot3(x_ref[...], w0_ref[...])
    t_ref[...] = h + b0_ref[...]
    h = jnp.maximum(_bn_ref(t_ref), 0.0)
    h = _dot3(h, w1_ref[...])
    o_ref[...] = h + b1_ref[...]


def _encoder(xp, w0p, b0, w1, b1):
    return pl.pallas_call(
        _enc_body,
        out_shape=jax.ShapeDtypeStruct((_N, _D), jnp.float32),
        scratch_shapes=[pltpu.VMEM((_N, _D), jnp.float32)],
    )(xp, w0p, b0, w1, b1)


def _eemb_body(ea_ref, w_ref, b_ref, o_ref):
    e = _dot3(ea_ref[...], w_ref[0])
    o_ref[0] = e + b_ref[0]


def _eemb(eap, wb, bb, blk):
    grid = (_L, _EPAD // blk)
    return pl.pallas_call(
        _eemb_body,
        grid=grid,
        in_specs=[
            pl.BlockSpec((blk, 16), lambda l, e: (e, 0)),
            pl.BlockSpec((1, 16, _D), lambda l, e: (l, 0, 0)),
            pl.BlockSpec((1, 1, _D), lambda l, e: (l, 0, 0)),
        ],
        out_specs=pl.BlockSpec((1, blk, _D), lambda l, e: (l, e, 0)),
        out_shape=jax.ShapeDtypeStruct((_L, _EPAD, _D), jnp.float32),
    )(eap, wb, bb)


def _mlp_body(h_ref, ag_ref, eps_ref, wa_ref, ba_ref, wb_ref, bb_ref, o_ref,
              t_ref, *, last):
    h = h_ref[...]
    z = (1.0 + eps_ref[0, 0]) * h + ag_ref[0, :_N, :] + ag_ref[1, :_N, :]
    z = _dot3(z, wa_ref[...])
    t_ref[...] = z + ba_ref[...]
    z = jnp.maximum(_bn_ref(t_ref), 0.0)
    z = _dot3(z, wb_ref[...])
    o_ref[...] = z + bb_ref[...]
    z = _bn_ref(o_ref)
    if not last:
        z = jnp.maximum(z, 0.0)
    o_ref[...] = z


def _node_mlp(h, aggr, eps, wa, ba, wb, bb, last):
    return pl.pallas_call(
        functools.partial(_mlp_body, last=last),
        out_shape=jax.ShapeDtypeStruct((_N, _D), jnp.float32),
        scratch_shapes=[pltpu.VMEM((_N, 2 * _D), jnp.float32)],
    )(h, aggr, eps, wa, ba, wb, bb)


def _sc_body(h_hbm, src_hbm, dst_hbm, eemb_hbm, out_hbm,
             idx_s, idx_d, buf_e, buf_h, aggr, sem):
    c = lax.axis_index("c")
    s = lax.axis_index("s")
    wid = c * _NS + s

    zero16 = jnp.zeros((16,), jnp.float32)

    def _zrow(j, carry):
        for k in range(_D // 16):
            buf_e[j, pl.ds(k * 16, 16)] = zero16
        return carry

    lax.fori_loop(0, _C, _zrow, 0)
    r0 = s * _RPT
    off = 0
    while off < _RPT:
        n = min(_C, _RPT - off)
        pltpu.sync_copy(buf_e.at[pl.ds(0, n)], aggr.at[pl.ds(r0 + off, n)])
        off += n
    plsc.subcore_barrier()

    base_e = wid * (_CH * _C)

    def _chunk(i, carry):
        base = base_e + i * _C
        pltpu.sync_copy(src_hbm.at[pl.ds(base, _C)], idx_s)
        pltpu.sync_copy(dst_hbm.at[pl.ds(base, _C)], idx_d)
        pltpu.sync_copy(eemb_hbm.at[pl.ds(base, _C)], buf_e)
        pltpu.async_copy(h_hbm.at[idx_s], buf_h, sem).wait()

        def _row(j, cc):
            for k in range(_D // 16):
                sl = pl.ds(k * 16, 16)
                buf_e[j, sl] = jnp.maximum(buf_e[j, sl] + buf_h[j, sl], 0.0)
            return cc

        lax.fori_loop(0, _C, _row, 0)
        pltpu.sync_copy(buf_e, aggr.at[idx_d], add=True)
        return carry

    lax.fori_loop(0, _CH, _chunk, 0)
    plsc.subcore_barrier()

    off = 0
    while off < _RPT:
        n = min(_C, _RPT - off)
        pltpu.sync_copy(aggr.at[pl.ds(r0 + off, n)],
                        out_hbm.at[c].at[pl.ds(r0 + off, n)])
        off += n


def _sc_message_pass(h, src, dst, eemb_l):
    mesh = plsc.VectorSubcoreMesh(
        core_axis_name="c", subcore_axis_name="s",
        num_cores=_NC, num_subcores=_NS)
    fn = pl.kernel(
        _sc_body,
        out_type=jax.ShapeDtypeStruct((_NC, _NPAD, _D), jnp.float32),
        mesh=mesh,
        scratch_types=[
            pltpu.VMEM((_C,), jnp.int32),
            pltpu.VMEM((_C,), jnp.int32),
            pltpu.VMEM((_C, _D), jnp.float32),
            pltpu.VMEM((_C, _D), jnp.float32),
            pltpu.VMEM_SHARED((_NPAD, _D), jnp.float32),
            pltpu.SemaphoreType.DMA,
        ],
    )
    return fn(h, src, dst, eemb_l)


def kernel(x, edge_index, edge_attr, batch, params):
    f32 = jnp.float32
    xp = jnp.pad(x, ((0, 0), (0, _D - _FIN)))
    w0p = jnp.pad(params['enc_W0'], ((0, _D - _FIN), (0, 0)))
    b0 = params['enc_b0'].reshape(1, _D)
    b1 = params['enc_b1'].reshape(1, _D)

    src = edge_index[0]
    dst = edge_index[1]
    pad_e = _EPAD - _E
    src_p = jnp.pad(src, (0, pad_e))
    dst_p = jnp.pad(dst, (0, pad_e), constant_values=_N)
    eap = jnp.pad(edge_attr, ((0, pad_e), (0, 16 - _FE)))

    wb = jnp.stack([jnp.pad(params['bond_W%d' % l], ((0, 16 - _FE), (0, 0)))
                    for l in range(_L)])
    bb = jnp.stack([params['bond_b%d' % l].reshape(1, _D) for l in range(_L)])

    h = _encoder(xp, w0p, b0, params['enc_W1'], b1)
    eemb = _eemb(eap, wb, bb, 4096)

    for l in range(_L):
        aggr = _sc_message_pass(h, src_p, dst_p, eemb[l])
        h = _node_mlp(
            h, aggr,
            params['eps%d' % l].reshape(1, 1),
            params['mlpA_W%d' % l], params['mlpA_b%d' % l].reshape(1, 2 * _D),
            params['mlpB_W%d' % l], params['mlpB_b%d' % l].reshape(1, _D),
            last=(l == _L - 1))
    return h
